# baseline (speedup 1.0000x reference)
import functools

import jax
import jax.numpy as jnp
from jax import lax
from jax.experimental import pallas as pl
from jax.experimental.pallas import tpu as pltpu
from jax.experimental.pallas import tpu_sc as plsc

BATCH = 16384
HIST = 50
EMBED = 64
HIST_P = 56

_NC = 2
_NS = 16
NW = _NC * _NS
ROWS_PER_W = BATCH // NW
R = 8
NCHUNK = ROWS_PER_W // R
NBUF = 2
CHUNK_I = R * HIST_P


def _gather_kernel(table_hbm, idxp_hbm, out_hbm, idx_v, *scratch):
    bufs = scratch[:NBUF]
    gsems = scratch[NBUF:2 * NBUF]
    ssems = scratch[2 * NBUF:]

    wid = lax.axis_index("s") * _NC + lax.axis_index("c")
    row0 = wid * ROWS_PER_W

    pltpu.sync_copy(idxp_hbm.at[pl.ds(row0 * HIST_P, ROWS_PER_W * HIST_P)],
                    idx_v)

    def gather_chunk(c, b):
        pltpu.async_copy(
            table_hbm.at[idx_v.at[pl.ds(c * CHUNK_I, CHUNK_I)]],
            bufs[b], gsems[b])

    def store_chunk(c, b):
        for j in range(R):
            pltpu.async_copy(bufs[b].at[pl.ds(j * HIST_P, HIST)],
                             out_hbm.at[row0 + c * R + j], ssems[b])

    def wait_gather(b):
        pltpu.make_async_copy(
            table_hbm.at[idx_v.at[pl.ds(0, CHUNK_I)]], bufs[b],
            gsems[b]).wait()

    def wait_stores(c, b):
        for j in range(R):
            pltpu.make_async_copy(bufs[b].at[pl.ds(j * HIST_P, HIST)],
                                  out_hbm.at[row0 + c * R + j],
                                  ssems[b]).wait()

    for b in range(NBUF):
        gather_chunk(b, b)

    def body(c0, carry):
        for b in range(NBUF):
            c = c0 + b
            wait_gather(b)
            store_chunk(c, b)

            @pl.when(c + NBUF < NCHUNK)
            def _():
                wait_stores(c, b)
                gather_chunk(c + NBUF, b)

        return carry

    lax.fori_loop(0, NCHUNK // NBUF, lambda g, c: body(g * NBUF, c), 0,
                  unroll=False)

    for b in range(NBUF):
        wait_stores(NCHUNK - NBUF + b, b)


def kernel(x, geometry):
    xp = jnp.pad(x, ((0, 0), (0, HIST_P - HIST))).reshape(BATCH * HIST_P)
    mesh = plsc.VectorSubcoreMesh(core_axis_name="c", subcore_axis_name="s")
    run = functools.partial(
        pl.kernel,
        mesh=mesh,
        out_type=jax.ShapeDtypeStruct((BATCH, HIST, EMBED), jnp.float32),
        scratch_types=(
            [pltpu.VMEM((ROWS_PER_W * HIST_P,), jnp.int32)]
            + [pltpu.VMEM((CHUNK_I, EMBED), jnp.float32) for _ in range(NBUF)]
            + [pltpu.SemaphoreType.DMA for _ in range(2 * NBUF)]
        ),
        compiler_params=pltpu.CompilerParams(use_tc_tiling_on_sc=False),
    )(_gather_kernel)
    return run(geometry, xp)


# speedup vs baseline: 3.0493x; 3.0493x over previous
import functools

import jax
import jax.numpy as jnp
from jax import lax
from jax.experimental import pallas as pl
from jax.experimental.pallas import tpu as pltpu
from jax.experimental.pallas import tpu_sc as plsc

BATCH = 16384
HIST = 50
EMBED = 64
EMBED_P = 128
B = BATCH * HIST

_NC = 2
_NS = 16
NW = _NC * _NS
B_PER_W = B // NW
CHUNK = 256
NCHUNK = B_PER_W // CHUNK
NBUF = 2
assert NCHUNK % NBUF == 0


def _gather_kernel(table_hbm, idx_hbm, out_hbm, idx_v, *scratch):
    bufs = scratch[:NBUF]
    gsems = scratch[NBUF:2 * NBUF]
    ssems = scratch[2 * NBUF:]

    wid = lax.axis_index("s") * _NC + lax.axis_index("c")
    base = wid * B_PER_W

    pltpu.sync_copy(idx_hbm.at[pl.ds(base, B_PER_W)], idx_v)

    def gather_chunk(c, b):
        pltpu.async_copy(
            table_hbm.at[idx_v.at[pl.ds(c * CHUNK, CHUNK)]],
            bufs[b], gsems[b])

    def wait_gather(b):
        pltpu.make_async_copy(
            table_hbm.at[idx_v.at[pl.ds(0, CHUNK)]], bufs[b], gsems[b]).wait()

    def store_chunk(c, b):
        pltpu.async_copy(bufs[b], out_hbm.at[pl.ds(base + c * CHUNK, CHUNK)],
                         ssems[b])

    def wait_store(c, b):
        pltpu.make_async_copy(bufs[b],
                              out_hbm.at[pl.ds(base + c * CHUNK, CHUNK)],
                              ssems[b]).wait()

    for b in range(NBUF):
        gather_chunk(b, b)

    def body(c0, carry):
        for b in range(NBUF):
            c = c0 + b
            wait_gather(b)
            store_chunk(c, b)

            @pl.when(c + NBUF < NCHUNK)
            def _():
                wait_store(c, b)
                gather_chunk(c + NBUF, b)

        return carry

    lax.fori_loop(0, NCHUNK // NBUF, lambda g, c: body(g * NBUF, c), 0,
                  unroll=False)

    for b in range(NBUF):
        wait_store(NCHUNK - NBUF + b, b)


def kernel(x, geometry):
    table_p = jnp.concatenate(
        [geometry, jnp.zeros((geometry.shape[0], EMBED_P - EMBED),
                             jnp.float32)], axis=1)
    mesh = plsc.VectorSubcoreMesh(core_axis_name="c", subcore_axis_name="s")
    run = functools.partial(
        pl.kernel,
        mesh=mesh,
        out_type=jax.ShapeDtypeStruct((B, EMBED_P), jnp.float32),
        scratch_types=(
            [pltpu.VMEM((B_PER_W,), jnp.int32)]
            + [pltpu.VMEM((CHUNK, EMBED_P), jnp.float32)
               for _ in range(NBUF)]
            + [pltpu.SemaphoreType.DMA for _ in range(2 * NBUF)]
        ),
    )(_gather_kernel)
    out_p = run(table_p, x.reshape(B))
    return out_p[:, :EMBED].reshape(BATCH, HIST, EMBED)


# speedup vs baseline: 3.6208x; 1.1874x over previous
import functools

import jax
import jax.numpy as jnp
from jax import lax
from jax.experimental import pallas as pl
from jax.experimental.pallas import tpu as pltpu
from jax.experimental.pallas import tpu_sc as plsc

BATCH = 16384
HIST = 50
EMBED = 64
NSPLIT = 2
BATCH_S = BATCH // NSPLIT
B = BATCH_S * HIST

_NC = 2
_NS = 16
NW = _NC * _NS
B_PER_W = B // NW
CHUNK = 256
NCHUNK = B_PER_W // CHUNK
NBUF = 2
assert NCHUNK % NBUF == 0


def _gather_kernel(table_hbm, idx_hbm, out_hbm, idx_v, *scratch):
    bufs = scratch[:NBUF]
    gsems = scratch[NBUF:2 * NBUF]
    ssems = scratch[2 * NBUF:]

    wid = lax.axis_index("s") * _NC + lax.axis_index("c")
    base = wid * B_PER_W

    pltpu.sync_copy(idx_hbm.at[pl.ds(base, B_PER_W)], idx_v)

    def gather_chunk(c, b):
        pltpu.async_copy(
            table_hbm.at[idx_v.at[pl.ds(c * CHUNK, CHUNK)]],
            bufs[b], gsems[b])

    def wait_gather(b):
        pltpu.make_async_copy(
            table_hbm.at[idx_v.at[pl.ds(0, CHUNK)]], bufs[b], gsems[b]).wait()

    def store_chunk(c, b):
        pltpu.async_copy(bufs[b], out_hbm.at[pl.ds(base + c * CHUNK, CHUNK)],
                         ssems[b])

    def wait_store(c, b):
        pltpu.make_async_copy(bufs[b],
                              out_hbm.at[pl.ds(base + c * CHUNK, CHUNK)],
                              ssems[b]).wait()

    for b in range(NBUF):
        gather_chunk(b, b)

    def body(c0, carry):
        for b in range(NBUF):
            c = c0 + b
            wait_gather(b)
            store_chunk(c, b)

            @pl.when(c + NBUF < NCHUNK)
            def _():
                wait_store(c, b)
                gather_chunk(c + NBUF, b)

        return carry

    lax.fori_loop(0, NCHUNK // NBUF, lambda g, c: body(g * NBUF, c), 0,
                  unroll=False)

    for b in range(NBUF):
        wait_store(NCHUNK - NBUF + b, b)


def kernel(x, geometry):
    mesh = plsc.VectorSubcoreMesh(core_axis_name="c", subcore_axis_name="s")
    run = functools.partial(
        pl.kernel,
        mesh=mesh,
        out_type=jax.ShapeDtypeStruct((B, EMBED), jnp.float32),
        scratch_types=(
            [pltpu.VMEM((B_PER_W,), jnp.int32)]
            + [pltpu.VMEM((CHUNK, EMBED), jnp.float32) for _ in range(NBUF)]
            + [pltpu.SemaphoreType.DMA for _ in range(2 * NBUF)]
        ),
        compiler_params=pltpu.CompilerParams(use_tc_tiling_on_sc=False),
    )(_gather_kernel)
    idx = x.reshape(BATCH * HIST)
    halves = [
        run(geometry, lax.dynamic_slice_in_dim(idx, s * B, B))
        .reshape(BATCH_S, HIST, EMBED)
        for s in range(NSPLIT)
    ]
    return jnp.concatenate(halves, axis=0)


# speedup vs baseline: 3.9884x; 1.1015x over previous
import functools

import jax
import jax.numpy as jnp
from jax import lax
from jax.experimental import pallas as pl
from jax.experimental.pallas import tpu as pltpu
from jax.experimental.pallas import tpu_sc as plsc

BATCH = 16384
HIST = 50
EMBED = 64
B = BATCH * HIST

_NC = 2
_NS = 16
NW = _NC * _NS
B_PER_W = B // NW
CHUNK = 512
NCHUNK = B_PER_W // CHUNK
NBUF = 2
assert NCHUNK % NBUF == 0


def _gather_kernel(table_hbm, idx_hbm, out_hbm, idx_v, *scratch):
    bufs = scratch[:NBUF]
    gsems = scratch[NBUF:2 * NBUF]
    ssems = scratch[2 * NBUF:]

    wid = lax.axis_index("s") * _NC + lax.axis_index("c")
    base = wid * B_PER_W

    pltpu.sync_copy(idx_hbm.at[pl.ds(base, B_PER_W)], idx_v)

    def gather_chunk(c, b):
        pltpu.async_copy(
            table_hbm.at[idx_v.at[pl.ds(c * CHUNK, CHUNK)]],
            bufs[b], gsems[b])

    def wait_gather(b):
        pltpu.make_async_copy(
            table_hbm.at[idx_v.at[pl.ds(0, CHUNK)]], bufs[b], gsems[b]).wait()

    def store_chunk(c, b):
        pltpu.async_copy(bufs[b], out_hbm.at[pl.ds(base + c * CHUNK, CHUNK)],
                         ssems[b])

    def wait_store(c, b):
        pltpu.make_async_copy(bufs[b],
                              out_hbm.at[pl.ds(base + c * CHUNK, CHUNK)],
                              ssems[b]).wait()

    for b in range(NBUF):
        gather_chunk(b, b)

    def body(c0, carry):
        for b in range(NBUF):
            c = c0 + b
            wait_gather(b)
            store_chunk(c, b)

            @pl.when(c + NBUF < NCHUNK)
            def _():
                wait_store(c, b)
                gather_chunk(c + NBUF, b)

        return carry

    lax.fori_loop(0, NCHUNK // NBUF, lambda g, c: body(g * NBUF, c), 0,
                  unroll=False)

    for b in range(NBUF):
        wait_store(NCHUNK - NBUF + b, b)


def kernel(x, geometry):
    mesh = plsc.VectorSubcoreMesh(core_axis_name="c", subcore_axis_name="s")
    run = functools.partial(
        pl.kernel,
        mesh=mesh,
        out_type=jax.ShapeDtypeStruct((B, EMBED), jnp.float32),
        scratch_types=(
            [pltpu.VMEM((B_PER_W,), jnp.int32)]
            + [pltpu.VMEM((CHUNK, EMBED), jnp.float32) for _ in range(NBUF)]
            + [pltpu.SemaphoreType.DMA for _ in range(2 * NBUF)]
        ),
        compiler_params=pltpu.CompilerParams(use_tc_tiling_on_sc=False),
    )(_gather_kernel)
    out = run(geometry, x.reshape(B))
    return out.reshape(BATCH, HIST, EMBED)
